# baseline (speedup 1.0000x reference)
import functools

import jax
import jax.numpy as jnp
from jax import lax
from jax.experimental import pallas as pl
from jax.experimental.pallas import tpu as pltpu
from jax.experimental.pallas import tpu_sc as plsc

N_NODES = 10000
NUM_HOSTS = 5000
ACC_ROWS = 10240
ROWS_PER_TILE = ACC_ROWS // 16
CHUNK = 128
EDGES_PER_TILE = 10240
N_CHUNKS = EDGES_PER_TILE // CHUNK
PADDED_EDGES = 2 * 16 * EDGES_PER_TILE
DEG_W = 16

_mesh = plsc.VectorSubcoreMesh(core_axis_name="c", subcore_axis_name="s")
_sc_params = pltpu.CompilerParams(use_tc_tiling_on_sc=False)


def _sc_degree_body(dst_hbm, zeros_hbm, ones_hbm, out_hbm,
                    didx_v, upd_v, bounce_v, acc_sh, sem):
    c = lax.axis_index("c")
    s = lax.axis_index("s")
    w = c * 16 + s
    pltpu.sync_copy(zeros_hbm, bounce_v)
    pltpu.sync_copy(ones_hbm, upd_v)
    pltpu.sync_copy(dst_hbm.at[w], didx_v)
    pltpu.sync_copy(bounce_v, acc_sh.at[pl.ds(s * ROWS_PER_TILE, ROWS_PER_TILE)])
    plsc.subcore_barrier()

    def step(k, carry):
        pltpu.sync_copy(upd_v, acc_sh.at[didx_v.at[k]], add=True)
        return carry

    lax.fori_loop(0, N_CHUNKS, step, 0)
    plsc.subcore_barrier()
    pltpu.sync_copy(acc_sh.at[pl.ds(s * ROWS_PER_TILE, ROWS_PER_TILE)], bounce_v)
    pltpu.sync_copy(bounce_v, out_hbm.at[c, pl.ds(s * ROWS_PER_TILE, ROWS_PER_TILE)])


_sc_degree = pl.kernel(
    _sc_degree_body,
    out_type=jax.ShapeDtypeStruct((2, ACC_ROWS, DEG_W), jnp.float32),
    mesh=_mesh,
    compiler_params=_sc_params,
    scratch_types=[
        pltpu.VMEM((N_CHUNKS, CHUNK), jnp.int32),
        pltpu.VMEM((CHUNK, DEG_W), jnp.float32),
        pltpu.VMEM((ROWS_PER_TILE, DEG_W), jnp.float32),
        pltpu.VMEM_SHARED((ACC_ROWS, DEG_W), jnp.float32),
        pltpu.SemaphoreType.DMA,
    ],
)


def _sc_prop_body(y_hbm, src_hbm, dst_hbm, zeros_hbm, out_hbm,
                  sidx_v, didx_v, rows_v, acc_sh, gsem0, gsem1, ssem0, ssem1):
    c = lax.axis_index("c")
    s = lax.axis_index("s")
    w = c * 16 + s

    init_ds = [
        pltpu.async_copy(
            zeros_hbm, acc_sh.at[pl.ds(s * ROWS_PER_TILE + j * CHUNK, CHUNK)], gsem0)
        for j in range(ROWS_PER_TILE // CHUNK)
    ]
    for d in init_ds:
        d.wait()
    plsc.subcore_barrier()

    half = N_CHUNKS // 2
    for h in range(2):
        pltpu.sync_copy(src_hbm.at[w, pl.ds(h * half, half)], sidx_v)
        pltpu.sync_copy(dst_hbm.at[w, pl.ds(h * half, half)], didx_v)

        def step(i, carry):
            k = 2 * i
            d0 = pltpu.async_copy(y_hbm.at[sidx_v.at[k]], rows_v.at[0], gsem0)
            d1 = pltpu.async_copy(y_hbm.at[sidx_v.at[k + 1]], rows_v.at[1], gsem1)
            d0.wait()
            pltpu.sync_copy(rows_v.at[0], acc_sh.at[didx_v.at[k]], add=True)
            d1.wait()
            pltpu.sync_copy(rows_v.at[1], acc_sh.at[didx_v.at[k + 1]], add=True)
            return carry

        lax.fori_loop(0, half // 2, step, 0)
    plsc.subcore_barrier()
    rb_ds = [
        pltpu.async_copy(
            acc_sh.at[pl.ds(s * ROWS_PER_TILE + j * CHUNK, CHUNK)],
            out_hbm.at[c, pl.ds(s * ROWS_PER_TILE + j * CHUNK, CHUNK)], gsem1)
        for j in range(ROWS_PER_TILE // CHUNK)
    ]
    for d in rb_ds:
        d.wait()


def _make_sc_prop(d):
    return pl.kernel(
        _sc_prop_body,
        out_type=jax.ShapeDtypeStruct((2, ACC_ROWS, d), jnp.float32),
        mesh=_mesh,
        compiler_params=_sc_params,
        scratch_types=[
            pltpu.VMEM((N_CHUNKS // 2, CHUNK), jnp.int32),
            pltpu.VMEM((N_CHUNKS // 2, CHUNK), jnp.int32),
            pltpu.VMEM((2, CHUNK, d), jnp.float32),
            pltpu.VMEM_SHARED((ACC_ROWS, d), jnp.float32),
            pltpu.SemaphoreType.DMA,
            pltpu.SemaphoreType.DMA,
            pltpu.SemaphoreType.DMA,
            pltpu.SemaphoreType.DMA,
        ],
    )


_sc_prop_128 = _make_sc_prop(128)
_sc_prop_64 = _make_sc_prop(64)


def _tc_prep_body(d0_ref, d1_ref, x_ref, dinv_ref, y1_ref):
    deg = d0_ref[...] + d1_ref[...] + 1.0
    dinv = lax.rsqrt(deg)
    dinv_ref[...] = dinv
    y1_ref[...] = x_ref[...] * dinv


def _tc_mid_body(p0_ref, p1_ref, x_ref, dinv_ref, w1_ref, b1_ref, w2_ref,
                 g_ref, y2_ref):
    dinv = dinv_ref[...]
    sx = (p0_ref[...] + p1_ref[...]) * dinv + x_ref[...] * (dinv * dinv)
    h = jnp.maximum(jnp.dot(sx, w1_ref[...],
                            preferred_element_type=jnp.float32) + b1_ref[...], 0.0)
    g = jnp.dot(h, w2_ref[...], preferred_element_type=jnp.float32)
    g_ref[...] = g
    y2_ref[...] = g * dinv


def _tc_head_body(p0_ref, p1_ref, g_ref, dinv_ref, b2_ref,
                  wo1_ref, bo1_ref, wo2_ref, bo2_ref, wo3_ref, bo3_ref,
                  probs_ref):
    dinv = dinv_ref[...]
    z = (p0_ref[...] + p1_ref[...]) * dinv + g_ref[...] * (dinv * dinv)
    z = jnp.maximum(z + b2_ref[...], 0.0)
    a = jnp.maximum(jnp.dot(z, wo1_ref[...],
                            preferred_element_type=jnp.float32) + bo1_ref[...], 0.0)
    a = jnp.maximum(jnp.dot(a, wo2_ref[...],
                            preferred_element_type=jnp.float32) + bo2_ref[...], 0.0)
    act = jnp.dot(a, wo3_ref[...], preferred_element_type=jnp.float32) + bo3_ref[...]
    m = jnp.max(act)
    e = jnp.exp(act - m)
    probs_ref[...] = e / jnp.sum(e)


def kernel(x, edge_index, W1, b1, W2, b2, Wo1, bo1, Wo2, bo2, Wo3, bo3):
    f32 = jnp.float32
    n = N_NODES
    pad = PADDED_EDGES - edge_index.shape[1]
    lane = jnp.arange(pad, dtype=jnp.int32) % 16
    src_p = jnp.concatenate([edge_index[0], lane]).reshape(32, N_CHUNKS, CHUNK)
    dst_p = jnp.concatenate([edge_index[1], n + lane]).reshape(32, N_CHUNKS, CHUNK)

    degs = _sc_degree(dst_p,
                      jnp.zeros((ROWS_PER_TILE, DEG_W), f32),
                      jnp.ones((CHUNK, DEG_W), f32))
    d0 = degs[0, :n, 0:1]
    d1 = degs[1, :n, 0:1]

    dinv, y1 = pl.pallas_call(
        _tc_prep_body,
        out_shape=(jax.ShapeDtypeStruct((n, 1), f32),
                   jax.ShapeDtypeStruct((n, 128), f32)),
    )(d0, d1, x)

    p1 = _sc_prop_128(y1, src_p, dst_p, jnp.zeros((CHUNK, 128), f32))

    g, y2 = pl.pallas_call(
        _tc_mid_body,
        out_shape=(jax.ShapeDtypeStruct((n, 64), f32),
                   jax.ShapeDtypeStruct((n, 64), f32)),
    )(p1[0, :n], p1[1, :n], x, dinv, W1, b1.reshape(1, -1), W2)

    p2 = _sc_prop_64(y2, src_p, dst_p, jnp.zeros((CHUNK, 64), f32))

    probs = pl.pallas_call(
        _tc_head_body,
        out_shape=jax.ShapeDtypeStruct((NUM_HOSTS, 10), f32),
    )(p2[0, :NUM_HOSTS], p2[1, :NUM_HOSTS], g[:NUM_HOSTS], dinv[:NUM_HOSTS],
      b2.reshape(1, -1), Wo1, bo1.reshape(1, -1), Wo2, bo2.reshape(1, -1),
      Wo3, bo3.reshape(1, -1))

    out = probs.T.reshape(1, NUM_HOSTS * 10)
    return jnp.concatenate([jnp.zeros((1, 2), f32), out], axis=1)


# speedup vs baseline: 1.0560x; 1.0560x over previous
import functools

import jax
import jax.numpy as jnp
from jax import lax
from jax.experimental import pallas as pl
from jax.experimental.pallas import tpu as pltpu
from jax.experimental.pallas import tpu_sc as plsc

N_NODES = 10000
NUM_HOSTS = 5000
ACC_ROWS = 10240
ROWS_PER_TILE = ACC_ROWS // 16
CHUNK = 128
EDGES_PER_TILE = 10240
N_CHUNKS = EDGES_PER_TILE // CHUNK
PADDED_EDGES = 2 * 16 * EDGES_PER_TILE
DEG_W = 16

_mesh = plsc.VectorSubcoreMesh(core_axis_name="c", subcore_axis_name="s")
_sc_params = pltpu.CompilerParams(use_tc_tiling_on_sc=False)


def _sc_degree_body(dst_hbm, zeros_hbm, ones_hbm, out_hbm,
                    didx_v, upd_v, bounce_v, acc_sh, sem):
    c = lax.axis_index("c")
    s = lax.axis_index("s")
    w = c * 16 + s
    pltpu.sync_copy(zeros_hbm, bounce_v)
    pltpu.sync_copy(ones_hbm, upd_v)
    pltpu.sync_copy(dst_hbm.at[w], didx_v)
    pltpu.sync_copy(bounce_v, acc_sh.at[pl.ds(s * ROWS_PER_TILE, ROWS_PER_TILE)])
    plsc.subcore_barrier()

    def step(k, carry):
        pltpu.sync_copy(upd_v, acc_sh.at[didx_v.at[k]], add=True)
        return carry

    lax.fori_loop(0, N_CHUNKS, step, 0)
    plsc.subcore_barrier()
    pltpu.sync_copy(acc_sh.at[pl.ds(s * ROWS_PER_TILE, ROWS_PER_TILE)], bounce_v)
    pltpu.sync_copy(bounce_v, out_hbm.at[c, pl.ds(s * ROWS_PER_TILE, ROWS_PER_TILE)])


_sc_degree = pl.kernel(
    _sc_degree_body,
    out_type=jax.ShapeDtypeStruct((2, ACC_ROWS, DEG_W), jnp.float32),
    mesh=_mesh,
    compiler_params=_sc_params,
    scratch_types=[
        pltpu.VMEM((N_CHUNKS, CHUNK), jnp.int32),
        pltpu.VMEM((CHUNK, DEG_W), jnp.float32),
        pltpu.VMEM((ROWS_PER_TILE, DEG_W), jnp.float32),
        pltpu.VMEM_SHARED((ACC_ROWS, DEG_W), jnp.float32),
        pltpu.SemaphoreType.DMA,
    ],
)


def _sc_prop_body(y_hbm, src_hbm, dst_hbm, zeros_hbm, out_hbm,
                  sidx_v, didx_v, rows_v, acc_sh, gsem0, gsem1, ssem0, ssem1):
    c = lax.axis_index("c")
    s = lax.axis_index("s")
    w = c * 16 + s

    pltpu.sync_copy(zeros_hbm, rows_v.at[0])
    for j in range(ROWS_PER_TILE // CHUNK):
        pltpu.sync_copy(rows_v.at[0], acc_sh.at[pl.ds(s * ROWS_PER_TILE + j * CHUNK, CHUNK)])
    plsc.subcore_barrier()

    half = N_CHUNKS // 2
    for h in range(2):
        pltpu.sync_copy(src_hbm.at[w, pl.ds(h * half, half)], sidx_v)
        pltpu.sync_copy(dst_hbm.at[w, pl.ds(h * half, half)], didx_v)

        def step(i, carry):
            k = 4 * i
            d0 = pltpu.async_copy(y_hbm.at[sidx_v.at[k]], rows_v.at[0], gsem0)
            d1 = pltpu.async_copy(y_hbm.at[sidx_v.at[k + 1]], rows_v.at[1], gsem1)
            d0.wait()
            s0 = pltpu.async_copy(rows_v.at[0], acc_sh.at[didx_v.at[k]], ssem0, add=True)
            d1.wait()
            s1 = pltpu.async_copy(rows_v.at[1], acc_sh.at[didx_v.at[k + 1]], ssem1, add=True)
            s0.wait()
            d2 = pltpu.async_copy(y_hbm.at[sidx_v.at[k + 2]], rows_v.at[0], gsem0)
            s1.wait()
            d3 = pltpu.async_copy(y_hbm.at[sidx_v.at[k + 3]], rows_v.at[1], gsem1)
            d2.wait()
            s2 = pltpu.async_copy(rows_v.at[0], acc_sh.at[didx_v.at[k + 2]], ssem0, add=True)
            d3.wait()
            s3 = pltpu.async_copy(rows_v.at[1], acc_sh.at[didx_v.at[k + 3]], ssem1, add=True)
            s2.wait()
            s3.wait()
            return carry

        lax.fori_loop(0, half // 4, step, 0)
    plsc.subcore_barrier()
    for j in range(ROWS_PER_TILE // CHUNK):
        r0 = s * ROWS_PER_TILE + j * CHUNK
        pltpu.sync_copy(acc_sh.at[pl.ds(r0, CHUNK)], rows_v.at[0])
        pltpu.sync_copy(rows_v.at[0], out_hbm.at[c, pl.ds(r0, CHUNK)])


def _make_sc_prop(d):
    return pl.kernel(
        _sc_prop_body,
        out_type=jax.ShapeDtypeStruct((2, ACC_ROWS, d), jnp.float32),
        mesh=_mesh,
        compiler_params=_sc_params,
        scratch_types=[
            pltpu.VMEM((N_CHUNKS // 2, CHUNK), jnp.int32),
            pltpu.VMEM((N_CHUNKS // 2, CHUNK), jnp.int32),
            pltpu.VMEM((2, CHUNK, d), jnp.float32),
            pltpu.VMEM_SHARED((ACC_ROWS, d), jnp.float32),
            pltpu.SemaphoreType.DMA,
            pltpu.SemaphoreType.DMA,
            pltpu.SemaphoreType.DMA,
            pltpu.SemaphoreType.DMA,
        ],
    )


_sc_prop_128 = _make_sc_prop(128)
_sc_prop_64 = _make_sc_prop(64)


def _tc_prep_body(d0_ref, d1_ref, x_ref, dinv_ref, y1_ref):
    deg = d0_ref[...] + d1_ref[...] + 1.0
    dinv = lax.rsqrt(deg)
    dinv_ref[...] = dinv
    y1_ref[...] = x_ref[...] * dinv


def _tc_mid_body(p0_ref, p1_ref, x_ref, dinv_ref, w1_ref, b1_ref, w2_ref,
                 g_ref, y2_ref):
    dinv = dinv_ref[...]
    sx = (p0_ref[...] + p1_ref[...]) * dinv + x_ref[...] * (dinv * dinv)
    h = jnp.maximum(jnp.dot(sx, w1_ref[...],
                            preferred_element_type=jnp.float32) + b1_ref[...], 0.0)
    g = jnp.dot(h, w2_ref[...], preferred_element_type=jnp.float32)
    g_ref[...] = g
    y2_ref[...] = g * dinv


def _tc_head_body(p0_ref, p1_ref, g_ref, dinv_ref, b2_ref,
                  wo1_ref, bo1_ref, wo2_ref, bo2_ref, wo3_ref, bo3_ref,
                  probs_ref):
    dinv = dinv_ref[...]
    z = (p0_ref[...] + p1_ref[...]) * dinv + g_ref[...] * (dinv * dinv)
    z = jnp.maximum(z + b2_ref[...], 0.0)
    a = jnp.maximum(jnp.dot(z, wo1_ref[...],
                            preferred_element_type=jnp.float32) + bo1_ref[...], 0.0)
    a = jnp.maximum(jnp.dot(a, wo2_ref[...],
                            preferred_element_type=jnp.float32) + bo2_ref[...], 0.0)
    act = jnp.dot(a, wo3_ref[...], preferred_element_type=jnp.float32) + bo3_ref[...]
    m = jnp.max(act)
    e = jnp.exp(act - m)
    probs_ref[...] = e / jnp.sum(e)


def kernel(x, edge_index, W1, b1, W2, b2, Wo1, bo1, Wo2, bo2, Wo3, bo3):
    f32 = jnp.float32
    n = N_NODES
    pad = PADDED_EDGES - edge_index.shape[1]
    lane = jnp.arange(pad, dtype=jnp.int32) % 16
    src_p = jnp.concatenate([edge_index[0], lane]).reshape(32, N_CHUNKS, CHUNK)
    dst_p = jnp.concatenate([edge_index[1], n + lane]).reshape(32, N_CHUNKS, CHUNK)

    degs = _sc_degree(dst_p,
                      jnp.zeros((ROWS_PER_TILE, DEG_W), f32),
                      jnp.ones((CHUNK, DEG_W), f32))
    d0 = degs[0, :n, 0:1]
    d1 = degs[1, :n, 0:1]

    dinv, y1 = pl.pallas_call(
        _tc_prep_body,
        out_shape=(jax.ShapeDtypeStruct((n, 1), f32),
                   jax.ShapeDtypeStruct((n, 128), f32)),
    )(d0, d1, x)

    p1 = _sc_prop_128(y1, src_p, dst_p, jnp.zeros((CHUNK, 128), f32))

    g, y2 = pl.pallas_call(
        _tc_mid_body,
        out_shape=(jax.ShapeDtypeStruct((n, 64), f32),
                   jax.ShapeDtypeStruct((n, 64), f32)),
    )(p1[0, :n], p1[1, :n], x, dinv, W1, b1.reshape(1, -1), W2)

    p2 = _sc_prop_64(y2, src_p, dst_p, jnp.zeros((CHUNK, 64), f32))

    probs = pl.pallas_call(
        _tc_head_body,
        out_shape=jax.ShapeDtypeStruct((NUM_HOSTS, 10), f32),
    )(p2[0, :NUM_HOSTS], p2[1, :NUM_HOSTS], g[:NUM_HOSTS], dinv[:NUM_HOSTS],
      b2.reshape(1, -1), Wo1, bo1.reshape(1, -1), Wo2, bo2.reshape(1, -1),
      Wo3, bo3.reshape(1, -1))

    out = probs.T.reshape(1, NUM_HOSTS * 10)
    return jnp.concatenate([jnp.zeros((1, 2), f32), out], axis=1)


# speedup vs baseline: 1.0799x; 1.0226x over previous
import functools

import jax
import jax.numpy as jnp
from jax import lax
from jax.experimental import pallas as pl
from jax.experimental.pallas import tpu as pltpu
from jax.experimental.pallas import tpu_sc as plsc

N_NODES = 10000
NUM_HOSTS = 5000
ACC_ROWS = 10240
ROWS_PER_TILE = ACC_ROWS // 16
CHUNK = 128
EDGES_PER_TILE = 10240
N_CHUNKS = EDGES_PER_TILE // CHUNK
PADDED_EDGES = 2 * 16 * EDGES_PER_TILE
DEG_W = 16

_mesh = plsc.VectorSubcoreMesh(core_axis_name="c", subcore_axis_name="s")
_sc_params = pltpu.CompilerParams(use_tc_tiling_on_sc=False)


def _sc_degree_body(dst_hbm, zeros_hbm, ones_hbm, out_hbm,
                    didx_v, upd_v, bounce_v, acc_sh, sem):
    c = lax.axis_index("c")
    s = lax.axis_index("s")
    w = c * 16 + s
    pltpu.sync_copy(zeros_hbm, bounce_v)
    pltpu.sync_copy(ones_hbm, upd_v)
    pltpu.sync_copy(dst_hbm.at[w], didx_v)
    pltpu.sync_copy(bounce_v, acc_sh.at[pl.ds(s * ROWS_PER_TILE, ROWS_PER_TILE)])
    plsc.subcore_barrier()

    def step(k, carry):
        pltpu.sync_copy(upd_v, acc_sh.at[didx_v.at[k]], add=True)
        return carry

    lax.fori_loop(0, N_CHUNKS, step, 0)
    plsc.subcore_barrier()
    pltpu.sync_copy(acc_sh.at[pl.ds(s * ROWS_PER_TILE, ROWS_PER_TILE)], bounce_v)
    pltpu.sync_copy(bounce_v, out_hbm.at[c, pl.ds(s * ROWS_PER_TILE, ROWS_PER_TILE)])


_sc_degree = pl.kernel(
    _sc_degree_body,
    out_type=jax.ShapeDtypeStruct((2, ACC_ROWS, DEG_W), jnp.float32),
    mesh=_mesh,
    compiler_params=_sc_params,
    scratch_types=[
        pltpu.VMEM((N_CHUNKS, CHUNK), jnp.int32),
        pltpu.VMEM((CHUNK, DEG_W), jnp.float32),
        pltpu.VMEM((ROWS_PER_TILE, DEG_W), jnp.float32),
        pltpu.VMEM_SHARED((ACC_ROWS, DEG_W), jnp.float32),
        pltpu.SemaphoreType.DMA,
    ],
)


def _sc_prop_body(y_hbm, src_hbm, dst_hbm, zeros_hbm, out_hbm,
                  sidx_v, didx_v, rows_v, acc_sh, gsem0, gsem1, ssem0, ssem1,
                  rb_tiles=16):
    c = lax.axis_index("c")
    s = lax.axis_index("s")
    w = c * 16 + s

    pltpu.sync_copy(zeros_hbm, rows_v.at[0])
    for j in range(ROWS_PER_TILE // CHUNK):
        pltpu.sync_copy(rows_v.at[0], acc_sh.at[pl.ds(s * ROWS_PER_TILE + j * CHUNK, CHUNK)])
    plsc.subcore_barrier()

    half = N_CHUNKS // 2
    for h in range(2):
        pltpu.sync_copy(src_hbm.at[w, pl.ds(h * half, half)], sidx_v)
        pltpu.sync_copy(dst_hbm.at[w, pl.ds(h * half, half)], didx_v)

        def step(i, carry):
            k = 4 * i
            d0 = pltpu.async_copy(y_hbm.at[sidx_v.at[k]], rows_v.at[0], gsem0)
            d1 = pltpu.async_copy(y_hbm.at[sidx_v.at[k + 1]], rows_v.at[1], gsem1)
            d0.wait()
            s0 = pltpu.async_copy(rows_v.at[0], acc_sh.at[didx_v.at[k]], ssem0, add=True)
            d1.wait()
            s1 = pltpu.async_copy(rows_v.at[1], acc_sh.at[didx_v.at[k + 1]], ssem1, add=True)
            s0.wait()
            d2 = pltpu.async_copy(y_hbm.at[sidx_v.at[k + 2]], rows_v.at[0], gsem0)
            s1.wait()
            d3 = pltpu.async_copy(y_hbm.at[sidx_v.at[k + 3]], rows_v.at[1], gsem1)
            d2.wait()
            s2 = pltpu.async_copy(rows_v.at[0], acc_sh.at[didx_v.at[k + 2]], ssem0, add=True)
            d3.wait()
            s3 = pltpu.async_copy(rows_v.at[1], acc_sh.at[didx_v.at[k + 3]], ssem1, add=True)
            s2.wait()
            s3.wait()
            return carry

        lax.fori_loop(0, half // 4, step, 0)
    plsc.subcore_barrier()

    @pl.when(s < rb_tiles)
    def _():
        for j in range(ROWS_PER_TILE // CHUNK):
            r0 = s * ROWS_PER_TILE + j * CHUNK
            pltpu.sync_copy(acc_sh.at[pl.ds(r0, CHUNK)], rows_v.at[0])
            pltpu.sync_copy(rows_v.at[0], out_hbm.at[c, pl.ds(r0, CHUNK)])


def _make_sc_prop(d, rb_tiles=16):
    return pl.kernel(
        functools.partial(_sc_prop_body, rb_tiles=rb_tiles),
        out_type=jax.ShapeDtypeStruct((2, rb_tiles * ROWS_PER_TILE, d), jnp.float32),
        mesh=_mesh,
        compiler_params=_sc_params,
        scratch_types=[
            pltpu.VMEM((N_CHUNKS // 2, CHUNK), jnp.int32),
            pltpu.VMEM((N_CHUNKS // 2, CHUNK), jnp.int32),
            pltpu.VMEM((2, CHUNK, d), jnp.float32),
            pltpu.VMEM_SHARED((ACC_ROWS, d), jnp.float32),
            pltpu.SemaphoreType.DMA,
            pltpu.SemaphoreType.DMA,
            pltpu.SemaphoreType.DMA,
            pltpu.SemaphoreType.DMA,
        ],
    )


_sc_prop_128 = _make_sc_prop(128)
_sc_prop_64 = _make_sc_prop(64, rb_tiles=8)


def _tc_prep_body(d0_ref, d1_ref, x_ref, dinv_ref, y1_ref):
    deg = d0_ref[...] + d1_ref[...] + 1.0
    dinv = lax.rsqrt(deg)
    dinv_ref[...] = dinv
    y1_ref[...] = x_ref[...] * dinv


def _tc_mid_body(p0_ref, p1_ref, x_ref, dinv_ref, w1_ref, b1_ref, w2_ref,
                 g_ref, y2_ref):
    dinv = dinv_ref[...]
    sx = (p0_ref[...] + p1_ref[...]) * dinv + x_ref[...] * (dinv * dinv)
    h = jnp.maximum(jnp.dot(sx, w1_ref[...],
                            preferred_element_type=jnp.float32) + b1_ref[...], 0.0)
    g = jnp.dot(h, w2_ref[...], preferred_element_type=jnp.float32)
    g_ref[...] = g
    y2_ref[...] = g * dinv


def _tc_head_body(p0_ref, p1_ref, g_ref, dinv_ref, b2_ref,
                  wo1_ref, bo1_ref, wo2_ref, bo2_ref, wo3_ref, bo3_ref,
                  probs_ref):
    dinv = dinv_ref[...]
    z = (p0_ref[...] + p1_ref[...]) * dinv + g_ref[...] * (dinv * dinv)
    z = jnp.maximum(z + b2_ref[...], 0.0)
    a = jnp.maximum(jnp.dot(z, wo1_ref[...],
                            preferred_element_type=jnp.float32) + bo1_ref[...], 0.0)
    a = jnp.maximum(jnp.dot(a, wo2_ref[...],
                            preferred_element_type=jnp.float32) + bo2_ref[...], 0.0)
    act = jnp.dot(a, wo3_ref[...], preferred_element_type=jnp.float32) + bo3_ref[...]
    m = jnp.max(act)
    e = jnp.exp(act - m)
    probs_ref[...] = e / jnp.sum(e)


def kernel(x, edge_index, W1, b1, W2, b2, Wo1, bo1, Wo2, bo2, Wo3, bo3):
    f32 = jnp.float32
    n = N_NODES
    pad = PADDED_EDGES - edge_index.shape[1]
    lane = jnp.arange(pad, dtype=jnp.int32) % 16
    src_p = jnp.concatenate([edge_index[0], lane]).reshape(32, N_CHUNKS, CHUNK)
    dst_p = jnp.concatenate([edge_index[1], n + lane]).reshape(32, N_CHUNKS, CHUNK)

    degs = _sc_degree(dst_p,
                      jnp.zeros((ROWS_PER_TILE, DEG_W), f32),
                      jnp.ones((CHUNK, DEG_W), f32))
    d0 = degs[0, :n, 0:1]
    d1 = degs[1, :n, 0:1]

    dinv, y1 = pl.pallas_call(
        _tc_prep_body,
        out_shape=(jax.ShapeDtypeStruct((n, 1), f32),
                   jax.ShapeDtypeStruct((n, 128), f32)),
    )(d0, d1, x)

    p1 = _sc_prop_128(y1, src_p, dst_p, jnp.zeros((CHUNK, 128), f32))

    g, y2 = pl.pallas_call(
        _tc_mid_body,
        out_shape=(jax.ShapeDtypeStruct((n, 64), f32),
                   jax.ShapeDtypeStruct((n, 64), f32)),
    )(p1[0, :n], p1[1, :n], x, dinv, W1, b1.reshape(1, -1), W2)

    p2 = _sc_prop_64(y2, src_p, dst_p, jnp.zeros((CHUNK, 64), f32))

    probs = pl.pallas_call(
        _tc_head_body,
        out_shape=jax.ShapeDtypeStruct((NUM_HOSTS, 10), f32),
    )(p2[0, :NUM_HOSTS], p2[1, :NUM_HOSTS], g[:NUM_HOSTS], dinv[:NUM_HOSTS],
      b2.reshape(1, -1), Wo1, bo1.reshape(1, -1), Wo2, bo2.reshape(1, -1),
      Wo3, bo3.reshape(1, -1))

    out = probs.T.reshape(1, NUM_HOSTS * 10)
    return jnp.concatenate([jnp.zeros((1, 2), f32), out], axis=1)


# speedup vs baseline: 1.0903x; 1.0097x over previous
import functools

import jax
import jax.numpy as jnp
from jax import lax
from jax.experimental import pallas as pl
from jax.experimental.pallas import tpu as pltpu
from jax.experimental.pallas import tpu_sc as plsc

N_NODES = 10000
NUM_HOSTS = 5000
ACC_ROWS = 10240
ROWS_PER_TILE = ACC_ROWS // 16
CHUNK = 128
EDGES_PER_TILE = 10240
N_CHUNKS = EDGES_PER_TILE // CHUNK
PADDED_EDGES = 2 * 16 * EDGES_PER_TILE
DEG_W = 16

_mesh = plsc.VectorSubcoreMesh(core_axis_name="c", subcore_axis_name="s")
_sc_params = pltpu.CompilerParams(use_tc_tiling_on_sc=False)


def _sc_degree_body(dst_hbm, zeros_hbm, ones_hbm, out_hbm,
                    didx_v, upd_v, bounce_v, acc_sh, sem, sem2):
    c = lax.axis_index("c")
    s = lax.axis_index("s")
    w = c * 16 + s
    pltpu.sync_copy(zeros_hbm, bounce_v)
    pltpu.sync_copy(ones_hbm, upd_v)
    pltpu.sync_copy(dst_hbm.at[w], didx_v)
    pltpu.sync_copy(bounce_v, acc_sh.at[pl.ds(s * ROWS_PER_TILE, ROWS_PER_TILE)])
    plsc.subcore_barrier()

    def step(i, carry):
        k = 2 * i
        s0 = pltpu.async_copy(upd_v, acc_sh.at[didx_v.at[k]], sem, add=True)
        s1 = pltpu.async_copy(upd_v, acc_sh.at[didx_v.at[k + 1]], sem2, add=True)
        s0.wait()
        s1.wait()
        return carry

    lax.fori_loop(0, N_CHUNKS // 2, step, 0)
    plsc.subcore_barrier()
    pltpu.sync_copy(acc_sh.at[pl.ds(s * ROWS_PER_TILE, ROWS_PER_TILE)], bounce_v)
    pltpu.sync_copy(bounce_v, out_hbm.at[c, pl.ds(s * ROWS_PER_TILE, ROWS_PER_TILE)])


_sc_degree = pl.kernel(
    _sc_degree_body,
    out_type=jax.ShapeDtypeStruct((2, ACC_ROWS, DEG_W), jnp.float32),
    mesh=_mesh,
    compiler_params=_sc_params,
    scratch_types=[
        pltpu.VMEM((N_CHUNKS, CHUNK), jnp.int32),
        pltpu.VMEM((CHUNK, DEG_W), jnp.float32),
        pltpu.VMEM((ROWS_PER_TILE, DEG_W), jnp.float32),
        pltpu.VMEM_SHARED((ACC_ROWS, DEG_W), jnp.float32),
        pltpu.SemaphoreType.DMA,
        pltpu.SemaphoreType.DMA,
    ],
)


def _sc_prop_body(y_hbm, src_hbm, dst_hbm, zeros_hbm, out_hbm,
                  sidx_v, didx_v, rows_v, acc_sh, gsem0, gsem1, ssem0, ssem1,
                  rb_tiles=16):
    c = lax.axis_index("c")
    s = lax.axis_index("s")
    w = c * 16 + s

    @pl.when(s < rb_tiles)
    def _():
        pltpu.sync_copy(zeros_hbm, rows_v.at[0])
        for j in range(ROWS_PER_TILE // CHUNK):
            pltpu.sync_copy(rows_v.at[0], acc_sh.at[pl.ds(s * ROWS_PER_TILE + j * CHUNK, CHUNK)])
    plsc.subcore_barrier()

    half = N_CHUNKS // 2
    for h in range(2):
        pltpu.sync_copy(src_hbm.at[w, pl.ds(h * half, half)], sidx_v)
        pltpu.sync_copy(dst_hbm.at[w, pl.ds(h * half, half)], didx_v)

        def step(i, carry):
            k = 4 * i
            d0 = pltpu.async_copy(y_hbm.at[sidx_v.at[k]], rows_v.at[0], gsem0)
            d1 = pltpu.async_copy(y_hbm.at[sidx_v.at[k + 1]], rows_v.at[1], gsem1)
            d0.wait()
            s0 = pltpu.async_copy(rows_v.at[0], acc_sh.at[didx_v.at[k]], ssem0, add=True)
            d1.wait()
            s1 = pltpu.async_copy(rows_v.at[1], acc_sh.at[didx_v.at[k + 1]], ssem1, add=True)
            s0.wait()
            d2 = pltpu.async_copy(y_hbm.at[sidx_v.at[k + 2]], rows_v.at[0], gsem0)
            s1.wait()
            d3 = pltpu.async_copy(y_hbm.at[sidx_v.at[k + 3]], rows_v.at[1], gsem1)
            d2.wait()
            s2 = pltpu.async_copy(rows_v.at[0], acc_sh.at[didx_v.at[k + 2]], ssem0, add=True)
            d3.wait()
            s3 = pltpu.async_copy(rows_v.at[1], acc_sh.at[didx_v.at[k + 3]], ssem1, add=True)
            s2.wait()
            s3.wait()
            return carry

        lax.fori_loop(0, half // 4, step, 0)
    plsc.subcore_barrier()

    @pl.when(s < rb_tiles)
    def _():
        for j in range(ROWS_PER_TILE // CHUNK):
            r0 = s * ROWS_PER_TILE + j * CHUNK
            pltpu.sync_copy(acc_sh.at[pl.ds(r0, CHUNK)], rows_v.at[0])
            pltpu.sync_copy(rows_v.at[0], out_hbm.at[c, pl.ds(r0, CHUNK)])


def _make_sc_prop(d, rb_tiles=16):
    return pl.kernel(
        functools.partial(_sc_prop_body, rb_tiles=rb_tiles),
        out_type=jax.ShapeDtypeStruct((2, rb_tiles * ROWS_PER_TILE, d), jnp.float32),
        mesh=_mesh,
        compiler_params=_sc_params,
        scratch_types=[
            pltpu.VMEM((N_CHUNKS // 2, CHUNK), jnp.int32),
            pltpu.VMEM((N_CHUNKS // 2, CHUNK), jnp.int32),
            pltpu.VMEM((2, CHUNK, d), jnp.float32),
            pltpu.VMEM_SHARED((ACC_ROWS, d), jnp.float32),
            pltpu.SemaphoreType.DMA,
            pltpu.SemaphoreType.DMA,
            pltpu.SemaphoreType.DMA,
            pltpu.SemaphoreType.DMA,
        ],
    )


_sc_prop_128 = _make_sc_prop(128)
_sc_prop_64 = _make_sc_prop(64, rb_tiles=8)


def _tc_prep_body(d0_ref, d1_ref, x_ref, dinv_ref, y1_ref):
    deg = d0_ref[...] + d1_ref[...] + 1.0
    dinv = lax.rsqrt(deg)
    dinv_ref[...] = dinv
    y1_ref[...] = x_ref[...] * dinv


def _tc_mid_body(p0_ref, p1_ref, x_ref, dinv_ref, w1_ref, b1_ref, w2_ref,
                 g_ref, y2_ref):
    dinv = dinv_ref[...]
    sx = (p0_ref[...] + p1_ref[...]) * dinv + x_ref[...] * (dinv * dinv)
    h = jnp.maximum(jnp.dot(sx, w1_ref[...],
                            preferred_element_type=jnp.float32) + b1_ref[...], 0.0)
    g = jnp.dot(h, w2_ref[...], preferred_element_type=jnp.float32)
    g_ref[...] = g
    y2_ref[...] = g * dinv


def _tc_head_body(p0_ref, p1_ref, g_ref, dinv_ref, b2_ref,
                  wo1_ref, bo1_ref, wo2_ref, bo2_ref, wo3_ref, bo3_ref,
                  probs_ref):
    dinv = dinv_ref[...]
    z = (p0_ref[...] + p1_ref[...]) * dinv + g_ref[...] * (dinv * dinv)
    z = jnp.maximum(z + b2_ref[...], 0.0)
    a = jnp.maximum(jnp.dot(z, wo1_ref[...],
                            preferred_element_type=jnp.float32) + bo1_ref[...], 0.0)
    a = jnp.maximum(jnp.dot(a, wo2_ref[...],
                            preferred_element_type=jnp.float32) + bo2_ref[...], 0.0)
    act = jnp.dot(a, wo3_ref[...], preferred_element_type=jnp.float32) + bo3_ref[...]
    m = jnp.max(act)
    e = jnp.exp(act - m)
    probs_ref[...] = e / jnp.sum(e)


def kernel(x, edge_index, W1, b1, W2, b2, Wo1, bo1, Wo2, bo2, Wo3, bo3):
    f32 = jnp.float32
    n = N_NODES
    pad = PADDED_EDGES - edge_index.shape[1]
    lane = jnp.arange(pad, dtype=jnp.int32) % 16
    src_p = jnp.concatenate([edge_index[0], lane]).reshape(32, N_CHUNKS, CHUNK)
    dst_p = jnp.concatenate([edge_index[1], n + lane]).reshape(32, N_CHUNKS, CHUNK)

    degs = _sc_degree(dst_p,
                      jnp.zeros((ROWS_PER_TILE, DEG_W), f32),
                      jnp.ones((CHUNK, DEG_W), f32))
    d0 = degs[0, :n, 0:1]
    d1 = degs[1, :n, 0:1]

    dinv, y1 = pl.pallas_call(
        _tc_prep_body,
        out_shape=(jax.ShapeDtypeStruct((n, 1), f32),
                   jax.ShapeDtypeStruct((n, 128), f32)),
    )(d0, d1, x)

    p1 = _sc_prop_128(y1, src_p, dst_p, jnp.zeros((CHUNK, 128), f32))

    g, y2 = pl.pallas_call(
        _tc_mid_body,
        out_shape=(jax.ShapeDtypeStruct((n, 64), f32),
                   jax.ShapeDtypeStruct((n, 64), f32)),
    )(p1[0, :n], p1[1, :n], x, dinv, W1, b1.reshape(1, -1), W2)

    p2 = _sc_prop_64(y2, src_p, dst_p, jnp.zeros((CHUNK, 64), f32))

    probs = pl.pallas_call(
        _tc_head_body,
        out_shape=jax.ShapeDtypeStruct((NUM_HOSTS, 10), f32),
    )(p2[0, :NUM_HOSTS], p2[1, :NUM_HOSTS], g[:NUM_HOSTS], dinv[:NUM_HOSTS],
      b2.reshape(1, -1), Wo1, bo1.reshape(1, -1), Wo2, bo2.reshape(1, -1),
      Wo3, bo3.reshape(1, -1))

    out = probs.T.reshape(1, NUM_HOSTS * 10)
    return jnp.concatenate([jnp.zeros((1, 2), f32), out], axis=1)


# speedup vs baseline: 1.1062x; 1.0146x over previous
import functools

import jax
import jax.numpy as jnp
from jax import lax
from jax.experimental import pallas as pl
from jax.experimental.pallas import tpu as pltpu
from jax.experimental.pallas import tpu_sc as plsc

N_NODES = 10000
NUM_HOSTS = 5000
ACC_ROWS = 10240
ROWS_PER_TILE = ACC_ROWS // 16
CHUNK = 128
EDGES_PER_TILE = 10240
N_CHUNKS = EDGES_PER_TILE // CHUNK
PADDED_EDGES = 2 * 16 * EDGES_PER_TILE
DEG_W = 16

_mesh = plsc.VectorSubcoreMesh(core_axis_name="c", subcore_axis_name="s")
_sc_params = pltpu.CompilerParams(use_tc_tiling_on_sc=False)


def _sc_degree_body(dst_hbm, zeros_hbm, ones_hbm, out_hbm,
                    didx_v, upd_v, bounce_v, acc_sh, sem, sem2):
    c = lax.axis_index("c")
    s = lax.axis_index("s")
    w = c * 16 + s
    pltpu.sync_copy(zeros_hbm, bounce_v)
    pltpu.sync_copy(ones_hbm, upd_v)
    pltpu.sync_copy(dst_hbm.at[w], didx_v)
    pltpu.sync_copy(bounce_v, acc_sh.at[pl.ds(s * ROWS_PER_TILE, ROWS_PER_TILE)])
    plsc.subcore_barrier()

    def step(i, carry):
        k = 2 * i
        s0 = pltpu.async_copy(upd_v, acc_sh.at[didx_v.at[k]], sem, add=True)
        s1 = pltpu.async_copy(upd_v, acc_sh.at[didx_v.at[k + 1]], sem2, add=True)
        s0.wait()
        s1.wait()
        return carry

    lax.fori_loop(0, N_CHUNKS // 2, step, 0)
    plsc.subcore_barrier()
    pltpu.sync_copy(acc_sh.at[pl.ds(s * ROWS_PER_TILE, ROWS_PER_TILE)], bounce_v)
    pltpu.sync_copy(bounce_v, out_hbm.at[c, pl.ds(s * ROWS_PER_TILE, ROWS_PER_TILE)])


_sc_degree = pl.kernel(
    _sc_degree_body,
    out_type=jax.ShapeDtypeStruct((2, ACC_ROWS, DEG_W), jnp.float32),
    mesh=_mesh,
    compiler_params=_sc_params,
    scratch_types=[
        pltpu.VMEM((N_CHUNKS, CHUNK), jnp.int32),
        pltpu.VMEM((CHUNK, DEG_W), jnp.float32),
        pltpu.VMEM((ROWS_PER_TILE, DEG_W), jnp.float32),
        pltpu.VMEM_SHARED((ACC_ROWS, DEG_W), jnp.float32),
        pltpu.SemaphoreType.DMA,
        pltpu.SemaphoreType.DMA,
    ],
)


def _sc_prop_body(y_hbm, src_hbm, dst_hbm, zeros_hbm, out_hbm,
                  sidx_v, didx_v, rows_v, acc_sh, gsem0, gsem1, ssem0, ssem1,
                  rb_tiles=16):
    c = lax.axis_index("c")
    s = lax.axis_index("s")
    w = c * 16 + s

    @pl.when(s < rb_tiles)
    def _():
        pltpu.sync_copy(zeros_hbm, rows_v.at[0])
        for j in range(ROWS_PER_TILE // CHUNK):
            pltpu.sync_copy(rows_v.at[0], acc_sh.at[pl.ds(s * ROWS_PER_TILE + j * CHUNK, CHUNK)])
    plsc.subcore_barrier()

    half = N_CHUNKS // 2
    for h in range(2):
        pltpu.sync_copy(src_hbm.at[w, pl.ds(h * half, half)], sidx_v)
        pltpu.sync_copy(dst_hbm.at[w, pl.ds(h * half, half)], didx_v)

        def step(i, carry):
            k = 4 * i
            d0 = pltpu.async_copy(y_hbm.at[sidx_v.at[k]], rows_v.at[0], gsem0)
            d1 = pltpu.async_copy(y_hbm.at[sidx_v.at[k + 1]], rows_v.at[1], gsem1)
            d0.wait()
            s0 = pltpu.async_copy(rows_v.at[0], acc_sh.at[didx_v.at[k]], ssem0, add=True)
            d1.wait()
            s1 = pltpu.async_copy(rows_v.at[1], acc_sh.at[didx_v.at[k + 1]], ssem1, add=True)
            s0.wait()
            d2 = pltpu.async_copy(y_hbm.at[sidx_v.at[k + 2]], rows_v.at[0], gsem0)
            s1.wait()
            d3 = pltpu.async_copy(y_hbm.at[sidx_v.at[k + 3]], rows_v.at[1], gsem1)
            d2.wait()
            s2 = pltpu.async_copy(rows_v.at[0], acc_sh.at[didx_v.at[k + 2]], ssem0, add=True)
            d3.wait()
            s3 = pltpu.async_copy(rows_v.at[1], acc_sh.at[didx_v.at[k + 3]], ssem1, add=True)
            s2.wait()
            s3.wait()
            return carry

        lax.fori_loop(0, half // 4, step, 0)
    plsc.subcore_barrier()

    @pl.when(s < rb_tiles)
    def _():
        for j in range(ROWS_PER_TILE // CHUNK):
            r0 = s * ROWS_PER_TILE + j * CHUNK
            pltpu.sync_copy(acc_sh.at[pl.ds(r0, CHUNK)], rows_v.at[0])
            pltpu.sync_copy(rows_v.at[0], out_hbm.at[c, pl.ds(r0, CHUNK)])


def _make_sc_prop(d, rb_tiles=16):
    return pl.kernel(
        functools.partial(_sc_prop_body, rb_tiles=rb_tiles),
        out_type=jax.ShapeDtypeStruct((2, rb_tiles * ROWS_PER_TILE, d), jnp.float32),
        mesh=_mesh,
        compiler_params=_sc_params,
        scratch_types=[
            pltpu.VMEM((N_CHUNKS // 2, CHUNK), jnp.int32),
            pltpu.VMEM((N_CHUNKS // 2, CHUNK), jnp.int32),
            pltpu.VMEM((2, CHUNK, d), jnp.float32),
            pltpu.VMEM_SHARED((ACC_ROWS, d), jnp.float32),
            pltpu.SemaphoreType.DMA,
            pltpu.SemaphoreType.DMA,
            pltpu.SemaphoreType.DMA,
            pltpu.SemaphoreType.DMA,
        ],
    )


def _sc_prop3_body(y_hbm, src_hbm, dst_hbm, zeros_hbm, out_hbm,
                   sidx_v, didx_v, rows_v, acc_sh, gsem0, gsem1, gsem2,
                   ssem0, ssem1, ssem2, rb_tiles=16):
    c = lax.axis_index("c")
    s = lax.axis_index("s")
    w = c * 16 + s

    @pl.when(s < rb_tiles)
    def _():
        pltpu.sync_copy(zeros_hbm, rows_v.at[0])
        for j in range(ROWS_PER_TILE // CHUNK):
            pltpu.sync_copy(rows_v.at[0], acc_sh.at[pl.ds(s * ROWS_PER_TILE + j * CHUNK, CHUNK)])
    pltpu.sync_copy(src_hbm.at[w], sidx_v)
    pltpu.sync_copy(dst_hbm.at[w], didx_v)
    plsc.subcore_barrier()

    gsems = (gsem0, gsem1, gsem2)
    ssems = (ssem0, ssem1, ssem2)

    def rotation(k, nc):
        ds = [pltpu.async_copy(y_hbm.at[sidx_v.at[k + j]], rows_v.at[j], gsems[j])
              for j in range(nc)]
        ss = []
        for j in range(nc):
            ds[j].wait()
            ss.append(pltpu.async_copy(rows_v.at[j], acc_sh.at[didx_v.at[k + j]],
                                       ssems[j], add=True))
        for j in range(nc):
            ss[j].wait()

    def step(i, carry):
        rotation(3 * i, 3)
        return carry

    lax.fori_loop(0, N_CHUNKS // 3, step, 0)
    rotation(N_CHUNKS - N_CHUNKS % 3, N_CHUNKS % 3)
    plsc.subcore_barrier()

    @pl.when(s < rb_tiles)
    def _():
        for j in range(ROWS_PER_TILE // CHUNK):
            r0 = s * ROWS_PER_TILE + j * CHUNK
            pltpu.sync_copy(acc_sh.at[pl.ds(r0, CHUNK)], rows_v.at[0])
            pltpu.sync_copy(rows_v.at[0], out_hbm.at[c, pl.ds(r0, CHUNK)])


def _make_sc_prop3(d, rb_tiles=16):
    return pl.kernel(
        functools.partial(_sc_prop3_body, rb_tiles=rb_tiles),
        out_type=jax.ShapeDtypeStruct((2, rb_tiles * ROWS_PER_TILE, d), jnp.float32),
        mesh=_mesh,
        compiler_params=_sc_params,
        scratch_types=[
            pltpu.VMEM((N_CHUNKS, CHUNK), jnp.int32),
            pltpu.VMEM((N_CHUNKS, CHUNK), jnp.int32),
            pltpu.VMEM((3, CHUNK, d), jnp.float32),
            pltpu.VMEM_SHARED((ACC_ROWS, d), jnp.float32),
            pltpu.SemaphoreType.DMA,
            pltpu.SemaphoreType.DMA,
            pltpu.SemaphoreType.DMA,
            pltpu.SemaphoreType.DMA,
            pltpu.SemaphoreType.DMA,
            pltpu.SemaphoreType.DMA,
        ],
    )


_sc_prop_128 = _make_sc_prop(128)
_sc_prop_64 = _make_sc_prop3(64, rb_tiles=8)


def _tc_prep_body(d0_ref, d1_ref, x_ref, dinv_ref, y1_ref):
    deg = d0_ref[...] + d1_ref[...] + 1.0
    dinv = lax.rsqrt(deg)
    dinv_ref[...] = dinv
    y1_ref[...] = x_ref[...] * dinv


def _tc_mid_body(p0_ref, p1_ref, x_ref, dinv_ref, w1_ref, b1_ref, w2_ref,
                 g_ref, y2_ref):
    dinv = dinv_ref[...]
    sx = (p0_ref[...] + p1_ref[...]) * dinv + x_ref[...] * (dinv * dinv)
    h = jnp.maximum(jnp.dot(sx, w1_ref[...],
                            preferred_element_type=jnp.float32) + b1_ref[...], 0.0)
    g = jnp.dot(h, w2_ref[...], preferred_element_type=jnp.float32)
    g_ref[...] = g
    y2_ref[...] = g * dinv


def _tc_head_body(p0_ref, p1_ref, g_ref, dinv_ref, b2_ref,
                  wo1_ref, bo1_ref, wo2_ref, bo2_ref, wo3_ref, bo3_ref,
                  probs_ref):
    dinv = dinv_ref[...]
    z = (p0_ref[...] + p1_ref[...]) * dinv + g_ref[...] * (dinv * dinv)
    z = jnp.maximum(z + b2_ref[...], 0.0)
    a = jnp.maximum(jnp.dot(z, wo1_ref[...],
                            preferred_element_type=jnp.float32) + bo1_ref[...], 0.0)
    a = jnp.maximum(jnp.dot(a, wo2_ref[...],
                            preferred_element_type=jnp.float32) + bo2_ref[...], 0.0)
    act = jnp.dot(a, wo3_ref[...], preferred_element_type=jnp.float32) + bo3_ref[...]
    m = jnp.max(act)
    e = jnp.exp(act - m)
    probs_ref[...] = e / jnp.sum(e)


def kernel(x, edge_index, W1, b1, W2, b2, Wo1, bo1, Wo2, bo2, Wo3, bo3):
    f32 = jnp.float32
    n = N_NODES
    pad = PADDED_EDGES - edge_index.shape[1]
    lane = jnp.arange(pad, dtype=jnp.int32) % 16
    src_p = jnp.concatenate([edge_index[0], lane]).reshape(32, N_CHUNKS, CHUNK)
    dst_p = jnp.concatenate([edge_index[1], n + lane]).reshape(32, N_CHUNKS, CHUNK)

    degs = _sc_degree(dst_p,
                      jnp.zeros((ROWS_PER_TILE, DEG_W), f32),
                      jnp.ones((CHUNK, DEG_W), f32))
    d0 = degs[0, :n, 0:1]
    d1 = degs[1, :n, 0:1]

    dinv, y1 = pl.pallas_call(
        _tc_prep_body,
        out_shape=(jax.ShapeDtypeStruct((n, 1), f32),
                   jax.ShapeDtypeStruct((n, 128), f32)),
    )(d0, d1, x)

    p1 = _sc_prop_128(y1, src_p, dst_p, jnp.zeros((CHUNK, 128), f32))

    g, y2 = pl.pallas_call(
        _tc_mid_body,
        out_shape=(jax.ShapeDtypeStruct((n, 64), f32),
                   jax.ShapeDtypeStruct((n, 64), f32)),
    )(p1[0, :n], p1[1, :n], x, dinv, W1, b1.reshape(1, -1), W2)

    p2 = _sc_prop_64(y2, src_p, dst_p, jnp.zeros((CHUNK, 64), f32))

    probs = pl.pallas_call(
        _tc_head_body,
        out_shape=jax.ShapeDtypeStruct((NUM_HOSTS, 10), f32),
    )(p2[0, :NUM_HOSTS], p2[1, :NUM_HOSTS], g[:NUM_HOSTS], dinv[:NUM_HOSTS],
      b2.reshape(1, -1), Wo1, bo1.reshape(1, -1), Wo2, bo2.reshape(1, -1),
      Wo3, bo3.reshape(1, -1))

    out = probs.T.reshape(1, NUM_HOSTS * 10)
    return jnp.concatenate([jnp.zeros((1, 2), f32), out], axis=1)
